# baseline (speedup 1.0000x reference)
import functools

import jax
import jax.numpy as jnp
from jax import lax
from jax.experimental import pallas as pl
from jax.experimental.pallas import tpu as pltpu
from jax.experimental.pallas import tpu_sc as plsc

VOCAB = 100000
EMBED_DIM = 32
GENRE_DIM = 19
BATCH = 16384
ROW_PACK = 128 // EMBED_DIM
PACK_STRIDE = 25088
PACK_ROWS = PACK_STRIDE
PACK_BLK = 512
CHUNK = 128


def _pack_body(t0_ref, t1_ref, t2_ref, t3_ref, out_ref):
  eye = (lax.broadcasted_iota(jnp.int32, (EMBED_DIM, EMBED_DIM), 0) ==
         lax.broadcasted_iota(jnp.int32, (EMBED_DIM, EMBED_DIM), 1)
         ).astype(jnp.float32)
  out_ref[...] = jnp.concatenate(
      [lax.dot_general(t_ref[...], eye, (((0,), (0,)), ((), ())),
                       preferred_element_type=jnp.float32)
       for t_ref in (t0_ref, t1_ref, t2_ref, t3_ref)], axis=1)


def _tc_pack(tableT):
  grid = (PACK_ROWS // PACK_BLK,)
  n_blk = PACK_STRIDE // PACK_BLK

  def spec(s):
    return pl.BlockSpec((EMBED_DIM, PACK_BLK), lambda i, s=s: (0, n_blk * s + i))

  return pl.pallas_call(
      _pack_body,
      grid=grid,
      in_specs=[spec(0), spec(1), spec(2), spec(3)],
      out_specs=pl.BlockSpec((PACK_BLK, 128), lambda i: (i, 0)),
      out_shape=jax.ShapeDtypeStruct((PACK_ROWS, 128), jnp.float32),
  )(tableT, tableT, tableT, tableT)


def _sc_gather(pack, movie_id):
  info = plsc.get_sparse_core_info()
  nc, ns, nl = info.num_cores, info.num_subcores, info.num_lanes
  nw = nc * ns
  b_per_w = BATCH // nw
  n_chunks = b_per_w // CHUNK

  mesh = plsc.VectorSubcoreMesh(core_axis_name="c", subcore_axis_name="s")

  def sgroup(v):
    s = (v >= PACK_STRIDE).astype(jnp.int32)
    s = s + (v >= 2 * PACK_STRIDE).astype(jnp.int32)
    return s + (v >= 3 * PACK_STRIDE).astype(jnp.int32)

  @functools.partial(
      pl.kernel,
      mesh=mesh,
      compiler_params=pltpu.CompilerParams(needs_layout_passes=False),
      out_type=jax.ShapeDtypeStruct((EMBED_DIM, BATCH), jnp.float32),
      scratch_types=[
          pltpu.VMEM((b_per_w,), jnp.int32),
          pltpu.VMEM((b_per_w,), jnp.int32),
          pltpu.VMEM((CHUNK, 128), jnp.float32),
          pltpu.VMEM((CHUNK, 128), jnp.float32),
          pltpu.VMEM((EMBED_DIM, b_per_w), jnp.float32),
          pltpu.SemaphoreType.DMA,
          pltpu.SemaphoreType.DMA,
      ],
  )
  def gather_kernel(pack_hbm, idx_hbm, out_hbm, idx_v, q_v, rows_a, rows_b,
                    outT_v, sem_a, sem_b):
    wid = lax.axis_index("s") * nc + lax.axis_index("c")
    base = wid * b_per_w
    pltpu.sync_copy(idx_hbm.at[pl.ds(base, b_per_w)], idx_v)
    for i in range(b_per_w // nl):
      v = idx_v[pl.ds(i * nl, nl)]
      q_v[pl.ds(i * nl, nl)] = v - sgroup(v) * PACK_STRIDE
    lane = lax.iota(jnp.int32, nl)
    bufs = [(rows_a, sem_a), (rows_b, sem_b)]

    def fire(c):
      rows_v, sem = bufs[c % 2]
      return pltpu.async_copy(
          pack_hbm.at[q_v.at[pl.ds(c * CHUNK, CHUNK)]], rows_v, sem)

    copies = {c: fire(c) for c in range(min(2, n_chunks))}
    for c in range(n_chunks):
      rows_v, _ = bufs[c % 2]
      copies[c].wait()
      def extract(jj, carry, c=c, rows_v=rows_v):
        col = c * CHUNK + jj * nl
        sv = idx_v[pl.ds(col, nl)]
        colbase = jax.lax.shift_left(sgroup(sv), 5)
        row = jj * nl + lane
        vals = [plsc.load_gather(rows_v, [row, colbase + d])
                for d in range(EMBED_DIM)]
        for d in range(EMBED_DIM):
          outT_v[d, pl.ds(col, nl)] = vals[d]
        return carry
      lax.fori_loop(0, CHUNK // nl, extract, 0)
      if c + 2 < n_chunks:
        copies[c + 2] = fire(c + 2)
    pltpu.sync_copy(outT_v, out_hbm.at[:, pl.ds(base, b_per_w)])

  return gather_kernel(pack, movie_id)


def _dotg(a, b, a_dim, b_dim):
  return lax.dot_general(
      a, b, (((a_dim,), (b_dim,)), ((), ())),
      preferred_element_type=jnp.float32)


def _mlp_body(meT_ref, gpT_ref, gW1_ref, gb1_ref, gW2T_ref, gb2_ref,
              rW1T_ref, rb1_ref, rW2T_ref, rb2_ref, rW3T_ref, rb3_ref,
              out_ref):
  gpT = gpT_ref[...]
  h = jnp.maximum(_dotg(gW1_ref[...], gpT, 0, 0) + gb1_ref[...], 0.0)
  geT = _dotg(gW2T_ref[...], h, 1, 0) + gb2_ref[...]
  combinedT = jnp.concatenate([meT_ref[...], geT], axis=0)
  h = jnp.maximum(_dotg(rW1T_ref[...], combinedT, 1, 0) + rb1_ref[...], 0.0)
  h = jnp.maximum(_dotg(rW2T_ref[...], h, 1, 0) + rb2_ref[...], 0.0)
  out_ref[...] = _dotg(rW3T_ref[...], h, 1, 0) + rb3_ref[...]


def _tc_mlp(movie_embT, gpT, gW1, gb1, gW2T, gb2,
            rW1T, rb1, rW2T, rb2, rW3T, rb3, block_b=2048):
  grid = (BATCH // block_b,)

  def full(a):
    return pl.BlockSpec(a.shape, lambda i: tuple(0 for _ in a.shape))

  return pl.pallas_call(
      _mlp_body,
      grid=grid,
      in_specs=[
          pl.BlockSpec((EMBED_DIM, block_b), lambda i: (0, i)),
          pl.BlockSpec((GENRE_DIM, block_b), lambda i: (0, i)),
          full(gW1), full(gb1), full(gW2T), full(gb2),
          full(rW1T), full(rb1), full(rW2T), full(rb2), full(rW3T), full(rb3),
      ],
      out_specs=pl.BlockSpec((1, block_b), lambda i: (0, i)),
      out_shape=jax.ShapeDtypeStruct((1, BATCH), jnp.float32),
  )(movie_embT, gpT, gW1, gb1, gW2T, gb2,
    rW1T, rb1, rW2T, rb2, rW3T, rb3)


@jax.jit
def kernel(movieId, genre_preferences, emb_table, gW1, gb1, gW2, gb2,
           rW1, rb1, rW2, rb2, rW3, rb3):
  pack = _tc_pack(emb_table.T)
  movie_embT = _sc_gather(pack, movieId.astype(jnp.int32))
  out = _tc_mlp(
      movie_embT, genre_preferences.T,
      gW1, gb1.reshape(-1, 1), gW2.T, gb2.reshape(-1, 1),
      rW1.T, rb1.reshape(-1, 1), rW2.T, rb2.reshape(-1, 1),
      rW3.T, rb3.reshape(1, 1))
  return out.reshape(BATCH, 1)


# speedup vs baseline: 1.2404x; 1.2404x over previous
import functools

import jax
import jax.numpy as jnp
from jax import lax
from jax.experimental import pallas as pl
from jax.experimental.pallas import tpu as pltpu
from jax.experimental.pallas import tpu_sc as plsc

VOCAB = 100000
EMBED_DIM = 32
GENRE_DIM = 19
BATCH = 16384
ROW_PACK = 128 // EMBED_DIM
PACK_STRIDE = 25088
PACK_ROWS = PACK_STRIDE
PACK_BLK = 3584
CHUNK = 128


def _pack_body(t0_ref, t1_ref, t2_ref, t3_ref, out_ref):
  eye = (lax.broadcasted_iota(jnp.int32, (EMBED_DIM, EMBED_DIM), 0) ==
         lax.broadcasted_iota(jnp.int32, (EMBED_DIM, EMBED_DIM), 1)
         ).astype(jnp.float32)
  out_ref[...] = jnp.concatenate(
      [lax.dot_general(t_ref[...], eye, (((0,), (0,)), ((), ())),
                       preferred_element_type=jnp.float32)
       for t_ref in (t0_ref, t1_ref, t2_ref, t3_ref)], axis=1)


def _tc_pack(tableT):
  grid = (PACK_ROWS // PACK_BLK,)
  n_blk = PACK_STRIDE // PACK_BLK

  def spec(s):
    return pl.BlockSpec((EMBED_DIM, PACK_BLK), lambda i, s=s: (0, n_blk * s + i))

  return pl.pallas_call(
      _pack_body,
      grid=grid,
      in_specs=[spec(0), spec(1), spec(2), spec(3)],
      out_specs=pl.BlockSpec((PACK_BLK, 128), lambda i: (i, 0)),
      out_shape=jax.ShapeDtypeStruct((PACK_ROWS, 128), jnp.float32),
  )(tableT, tableT, tableT, tableT)


def _sc_gather(pack, movie_id):
  info = plsc.get_sparse_core_info()
  nc, ns, nl = info.num_cores, info.num_subcores, info.num_lanes
  nw = nc * ns
  b_per_w = BATCH // nw
  n_chunks = b_per_w // CHUNK

  mesh = plsc.VectorSubcoreMesh(core_axis_name="c", subcore_axis_name="s")

  def sgroup(v):
    s = (v >= PACK_STRIDE).astype(jnp.int32)
    s = s + (v >= 2 * PACK_STRIDE).astype(jnp.int32)
    return s + (v >= 3 * PACK_STRIDE).astype(jnp.int32)

  @functools.partial(
      pl.kernel,
      mesh=mesh,
      compiler_params=pltpu.CompilerParams(needs_layout_passes=False),
      out_type=jax.ShapeDtypeStruct((EMBED_DIM, BATCH), jnp.float32),
      scratch_types=[
          pltpu.VMEM((b_per_w,), jnp.int32),
          pltpu.VMEM((b_per_w,), jnp.int32),
          pltpu.VMEM((CHUNK, 128), jnp.float32),
          pltpu.VMEM((CHUNK, 128), jnp.float32),
          pltpu.VMEM((EMBED_DIM, b_per_w), jnp.float32),
          pltpu.SemaphoreType.DMA,
          pltpu.SemaphoreType.DMA,
      ],
  )
  def gather_kernel(pack_hbm, idx_hbm, out_hbm, idx_v, q_v, rows_a, rows_b,
                    outT_v, sem_a, sem_b):
    wid = lax.axis_index("s") * nc + lax.axis_index("c")
    base = wid * b_per_w
    pltpu.sync_copy(idx_hbm.at[pl.ds(base, b_per_w)], idx_v)
    for i in range(b_per_w // nl):
      v = idx_v[pl.ds(i * nl, nl)]
      q_v[pl.ds(i * nl, nl)] = v - sgroup(v) * PACK_STRIDE
    lane = lax.iota(jnp.int32, nl)
    bufs = [(rows_a, sem_a), (rows_b, sem_b)]

    def fire(c):
      rows_v, sem = bufs[c % 2]
      return pltpu.async_copy(
          pack_hbm.at[q_v.at[pl.ds(c * CHUNK, CHUNK)]], rows_v, sem)

    copies = {c: fire(c) for c in range(min(2, n_chunks))}
    for c in range(n_chunks):
      rows_v, _ = bufs[c % 2]
      copies[c].wait()
      def extract(jj, carry, c=c, rows_v=rows_v):
        col = c * CHUNK + jj * nl
        sv = idx_v[pl.ds(col, nl)]
        colbase = jax.lax.shift_left(sgroup(sv), 5)
        row = jj * nl + lane
        vals = [plsc.load_gather(rows_v, [row, colbase + d])
                for d in range(EMBED_DIM)]
        for d in range(EMBED_DIM):
          outT_v[d, pl.ds(col, nl)] = vals[d]
        return carry
      lax.fori_loop(0, CHUNK // nl, extract, 0)
      if c + 2 < n_chunks:
        copies[c + 2] = fire(c + 2)
    pltpu.sync_copy(outT_v, out_hbm.at[:, pl.ds(base, b_per_w)])

  return gather_kernel(pack, movie_id)


def _dotg(a, b, a_dim, b_dim):
  return lax.dot_general(
      a, b, (((a_dim,), (b_dim,)), ((), ())),
      preferred_element_type=jnp.float32)


def _mlp_body(meT_ref, gpT_ref, gW1_ref, gb1_ref, gW2T_ref, gb2_ref,
              rW1T_ref, rb1_ref, rW2T_ref, rb2_ref, rW3T_ref, rb3_ref,
              out_ref):
  gpT = gpT_ref[...]
  h = jnp.maximum(_dotg(gW1_ref[...], gpT, 0, 0) + gb1_ref[...], 0.0)
  geT = _dotg(gW2T_ref[...], h, 1, 0) + gb2_ref[...]
  combinedT = jnp.concatenate([meT_ref[...], geT], axis=0)
  h = jnp.maximum(_dotg(rW1T_ref[...], combinedT, 1, 0) + rb1_ref[...], 0.0)
  h = jnp.maximum(_dotg(rW2T_ref[...], h, 1, 0) + rb2_ref[...], 0.0)
  out_ref[...] = _dotg(rW3T_ref[...], h, 1, 0) + rb3_ref[...]


def _tc_mlp(movie_embT, gpT, gW1, gb1, gW2T, gb2,
            rW1T, rb1, rW2T, rb2, rW3T, rb3, block_b=2048):
  grid = (BATCH // block_b,)

  def full(a):
    return pl.BlockSpec(a.shape, lambda i: tuple(0 for _ in a.shape))

  return pl.pallas_call(
      _mlp_body,
      grid=grid,
      in_specs=[
          pl.BlockSpec((EMBED_DIM, block_b), lambda i: (0, i)),
          pl.BlockSpec((GENRE_DIM, block_b), lambda i: (0, i)),
          full(gW1), full(gb1), full(gW2T), full(gb2),
          full(rW1T), full(rb1), full(rW2T), full(rb2), full(rW3T), full(rb3),
      ],
      out_specs=pl.BlockSpec((1, block_b), lambda i: (0, i)),
      out_shape=jax.ShapeDtypeStruct((1, BATCH), jnp.float32),
  )(movie_embT, gpT, gW1, gb1, gW2T, gb2,
    rW1T, rb1, rW2T, rb2, rW3T, rb3)


@jax.jit
def kernel(movieId, genre_preferences, emb_table, gW1, gb1, gW2, gb2,
           rW1, rb1, rW2, rb2, rW3, rb3):
  pack = _tc_pack(emb_table.T)
  movie_embT = _sc_gather(pack, movieId.astype(jnp.int32))
  out = _tc_mlp(
      movie_embT, genre_preferences.T,
      gW1, gb1.reshape(-1, 1), gW2.T, gb2.reshape(-1, 1),
      rW1.T, rb1.reshape(-1, 1), rW2.T, rb2.reshape(-1, 1),
      rW3.T, rb3.reshape(1, 1))
  return out.reshape(BATCH, 1)


# speedup vs baseline: 1.2979x; 1.0463x over previous
import functools

import jax
import jax.numpy as jnp
from jax import lax
from jax.experimental import pallas as pl
from jax.experimental.pallas import tpu as pltpu
from jax.experimental.pallas import tpu_sc as plsc

VOCAB = 100000
EMBED_DIM = 32
GENRE_DIM = 19
BATCH = 16384
ROW_PACK = 128 // EMBED_DIM
PACK_STRIDE = 25088
PACK_ROWS = PACK_STRIDE
PACK_BLK = 6272
CHUNK = 128


def _pack_body(t0_ref, t1_ref, t2_ref, t3_ref, out_ref):
  eye = (lax.broadcasted_iota(jnp.int32, (EMBED_DIM, EMBED_DIM), 0) ==
         lax.broadcasted_iota(jnp.int32, (EMBED_DIM, EMBED_DIM), 1)
         ).astype(jnp.float32)
  out_ref[...] = jnp.concatenate(
      [lax.dot_general(t_ref[...], eye, (((0,), (0,)), ((), ())),
                       preferred_element_type=jnp.float32)
       for t_ref in (t0_ref, t1_ref, t2_ref, t3_ref)], axis=1)


def _tc_pack(tableT):
  grid = (PACK_ROWS // PACK_BLK,)
  n_blk = PACK_STRIDE // PACK_BLK

  def spec(s):
    return pl.BlockSpec((EMBED_DIM, PACK_BLK), lambda i, s=s: (0, n_blk * s + i))

  return pl.pallas_call(
      _pack_body,
      grid=grid,
      in_specs=[spec(0), spec(1), spec(2), spec(3)],
      out_specs=pl.BlockSpec((PACK_BLK, 128), lambda i: (i, 0)),
      out_shape=jax.ShapeDtypeStruct((PACK_ROWS, 128), jnp.float32),
  )(tableT, tableT, tableT, tableT)


def _sc_gather(pack, movie_id):
  info = plsc.get_sparse_core_info()
  nc, ns, nl = info.num_cores, info.num_subcores, info.num_lanes
  nw = nc * ns
  b_per_w = BATCH // nw
  n_chunks = b_per_w // CHUNK

  mesh = plsc.VectorSubcoreMesh(core_axis_name="c", subcore_axis_name="s")

  def sgroup(v):
    s = (v >= PACK_STRIDE).astype(jnp.int32)
    s = s + (v >= 2 * PACK_STRIDE).astype(jnp.int32)
    return s + (v >= 3 * PACK_STRIDE).astype(jnp.int32)

  @functools.partial(
      pl.kernel,
      mesh=mesh,
      compiler_params=pltpu.CompilerParams(needs_layout_passes=False),
      out_type=jax.ShapeDtypeStruct((EMBED_DIM, BATCH), jnp.float32),
      scratch_types=[
          pltpu.VMEM((b_per_w,), jnp.int32),
          pltpu.VMEM((b_per_w,), jnp.int32),
          pltpu.VMEM((CHUNK, 128), jnp.float32),
          pltpu.VMEM((CHUNK, 128), jnp.float32),
          pltpu.VMEM((EMBED_DIM, b_per_w), jnp.float32),
          pltpu.SemaphoreType.DMA,
          pltpu.SemaphoreType.DMA,
      ],
  )
  def gather_kernel(pack_hbm, idx_hbm, out_hbm, idx_v, q_v, rows_a, rows_b,
                    outT_v, sem_a, sem_b):
    wid = lax.axis_index("s") * nc + lax.axis_index("c")
    base = wid * b_per_w
    pltpu.sync_copy(idx_hbm.at[pl.ds(base, b_per_w)], idx_v)
    for i in range(b_per_w // nl):
      v = idx_v[pl.ds(i * nl, nl)]
      q_v[pl.ds(i * nl, nl)] = v - sgroup(v) * PACK_STRIDE
    lane = lax.iota(jnp.int32, nl)
    bufs = [(rows_a, sem_a), (rows_b, sem_b)]

    def fire(c):
      rows_v, sem = bufs[c % 2]
      return pltpu.async_copy(
          pack_hbm.at[q_v.at[pl.ds(c * CHUNK, CHUNK)]], rows_v, sem)

    copies = {c: fire(c) for c in range(min(2, n_chunks))}
    for c in range(n_chunks):
      rows_v, _ = bufs[c % 2]
      copies[c].wait()
      def extract(jj, carry, c=c, rows_v=rows_v):
        col = c * CHUNK + jj * nl
        sv = idx_v[pl.ds(col, nl)]
        colbase = jax.lax.shift_left(sgroup(sv), 5)
        row = jj * nl + lane
        vals = [plsc.load_gather(rows_v, [row, colbase + d])
                for d in range(EMBED_DIM)]
        for d in range(EMBED_DIM):
          outT_v[d, pl.ds(col, nl)] = vals[d]
        return carry
      lax.fori_loop(0, CHUNK // nl, extract, 0)
      if c + 2 < n_chunks:
        copies[c + 2] = fire(c + 2)
    pltpu.sync_copy(outT_v, out_hbm.at[:, pl.ds(base, b_per_w)])

  return gather_kernel(pack, movie_id)


def _dotg(a, b, a_dim, b_dim):
  return lax.dot_general(
      a, b, (((a_dim,), (b_dim,)), ((), ())),
      preferred_element_type=jnp.float32)


def _mlp_body(meT_ref, gpT_ref, gW1_ref, gb1_ref, gW2T_ref, gb2_ref,
              rW1T_ref, rb1_ref, rW2T_ref, rb2_ref, rW3T_ref, rb3_ref,
              out_ref):
  gpT = gpT_ref[...]
  h = jnp.maximum(_dotg(gW1_ref[...], gpT, 0, 0) + gb1_ref[...], 0.0)
  geT = _dotg(gW2T_ref[...], h, 1, 0) + gb2_ref[...]
  combinedT = jnp.concatenate([meT_ref[...], geT], axis=0)
  h = jnp.maximum(_dotg(rW1T_ref[...], combinedT, 1, 0) + rb1_ref[...], 0.0)
  h = jnp.maximum(_dotg(rW2T_ref[...], h, 1, 0) + rb2_ref[...], 0.0)
  out_ref[...] = _dotg(rW3T_ref[...], h, 1, 0) + rb3_ref[...]


def _tc_mlp(movie_embT, gpT, gW1, gb1, gW2T, gb2,
            rW1T, rb1, rW2T, rb2, rW3T, rb3, block_b=4096):
  grid = (BATCH // block_b,)

  def full(a):
    return pl.BlockSpec(a.shape, lambda i: tuple(0 for _ in a.shape))

  return pl.pallas_call(
      _mlp_body,
      grid=grid,
      in_specs=[
          pl.BlockSpec((EMBED_DIM, block_b), lambda i: (0, i)),
          pl.BlockSpec((GENRE_DIM, block_b), lambda i: (0, i)),
          full(gW1), full(gb1), full(gW2T), full(gb2),
          full(rW1T), full(rb1), full(rW2T), full(rb2), full(rW3T), full(rb3),
      ],
      out_specs=pl.BlockSpec((1, block_b), lambda i: (0, i)),
      out_shape=jax.ShapeDtypeStruct((1, BATCH), jnp.float32),
  )(movie_embT, gpT, gW1, gb1, gW2T, gb2,
    rW1T, rb1, rW2T, rb2, rW3T, rb3)


@jax.jit
def kernel(movieId, genre_preferences, emb_table, gW1, gb1, gW2, gb2,
           rW1, rb1, rW2, rb2, rW3, rb3):
  pack = _tc_pack(emb_table.T)
  movie_embT = _sc_gather(pack, movieId.astype(jnp.int32))
  out = _tc_mlp(
      movie_embT, genre_preferences.T,
      gW1, gb1.reshape(-1, 1), gW2.T, gb2.reshape(-1, 1),
      rW1.T, rb1.reshape(-1, 1), rW2.T, rb2.reshape(-1, 1),
      rW3.T, rb3.reshape(1, 1))
  return out.reshape(BATCH, 1)
